# baseline (speedup 1.0000x reference)
import functools

import jax
import jax.numpy as jnp
from jax import lax
from jax.experimental import pallas as pl
from jax.experimental.pallas import tpu as pltpu
from jax.experimental.pallas import tpu_sc as plsc

K = 50
N_NEG = 16384
N_POS = 4096
NC = 1
NS = 16
NW = NC * NS
CHUNK = N_NEG // NW
SLOTS = 64
LANES = 16


def _sc_topk_body(neg_hbm, vals_hbm, wts_hbm, chunk_v, key_v, vals_v, wts_v):
    wid = lax.axis_index("s") * NC + lax.axis_index("c")
    base = wid * CHUNK
    pltpu.sync_copy(neg_hbm.at[pl.ds(base, CHUNK)], chunk_v)

    neg_inf = jnp.float32(-jnp.inf)
    msb = jnp.int32(-2147483648)
    low31 = jnp.int32(2147483647)
    ones = jnp.ones((LANES,), jnp.float32)
    zeros = jnp.zeros((LANES,), jnp.float32)

    for i in range(CHUNK // LANES):
        b = plsc.bitcast(chunk_v[pl.ds(i * LANES, LANES)], jnp.int32)
        key_v[pl.ds(i * LANES, LANES)] = b ^ (
            lax.shift_right_arithmetic(b, 31) & low31)

    def rnd(r, p):
        bit = lax.shift_left(jnp.int32(1), 31 - r)
        t_u = p | bit
        t_s = t_u ^ msb
        cv = zeros
        for i in range(CHUNK // LANES):
            kk = key_v[pl.ds(i * LANES, LANES)]
            cv = cv + jnp.where(kk >= t_s, ones, zeros)
        return jnp.where(jnp.sum(cv) >= jnp.float32(K), t_u, p)

    t_s = lax.fori_loop(0, 32, rnd, jnp.int32(0)) ^ msb

    for g in range(SLOTS // LANES):
        vals_v[pl.ds(g * LANES, LANES)] = jnp.full((LANES,), neg_inf, jnp.float32)

    def emit(i, off):
        x = chunk_v[pl.ds(i * LANES, LANES)]
        sel = key_v[pl.ds(i * LANES, LANES)] > t_s
        ind = jnp.where(sel, jnp.int32(1), jnp.int32(0))
        incl = plsc.cumsum(ind)
        plsc.store_scatter(vals_v, [off + incl - ind], x, mask=sel)
        return off + jnp.sum(ind)

    off = lax.fori_loop(0, CHUNK // LANES, emit, jnp.int32(0))

    lane0 = lax.iota(jnp.int32, LANES) == 0
    t_vec = jnp.full((LANES,), t_s, jnp.int32)
    t_f = plsc.bitcast(
        t_vec ^ (lax.shift_right_arithmetic(t_vec, 31) & low31), jnp.float32)
    plsc.store_scatter(vals_v, [jnp.full((LANES,), off, jnp.int32)], t_f, mask=lane0)
    w_t = jnp.float32(K) - off.astype(jnp.float32)
    for g in range(SLOTS // LANES):
        idx16 = lax.iota(jnp.int32, LANES) + g * LANES
        wts_v[pl.ds(g * LANES, LANES)] = (
            jnp.where(idx16 < off, ones, zeros)
            + jnp.where(idx16 == off, jnp.full((LANES,), w_t, jnp.float32), zeros))
        v = vals_v[pl.ds(g * LANES, LANES)]
        vals_v[pl.ds(g * LANES, LANES)] = jnp.where(idx16 > off, neg_inf, v)

    pltpu.sync_copy(vals_v, vals_hbm.at[pl.ds(wid * SLOTS, SLOTS)])
    pltpu.sync_copy(wts_v, wts_hbm.at[pl.ds(wid * SLOTS, SLOTS)])


@functools.cache
def _sc_topk():
    return pl.kernel(
        _sc_topk_body,
        mesh=plsc.VectorSubcoreMesh(
            core_axis_name="c", subcore_axis_name="s",
            num_cores=NC, num_subcores=NS),
        out_type=(
            jax.ShapeDtypeStruct((NW * SLOTS,), jnp.float32),
            jax.ShapeDtypeStruct((NW * SLOTS,), jnp.float32),
        ),
        scratch_types=(
            pltpu.VMEM((CHUNK,), jnp.float32),
            pltpu.VMEM((CHUNK,), jnp.int32),
            pltpu.VMEM((SLOTS,), jnp.float32),
            pltpu.VMEM((SLOTS,), jnp.float32),
        ),
        compiler_params=pltpu.CompilerParams(needs_layout_passes=False),
    )


def _tc_reduce_body(vals_ref, wts_ref, pos_ref, out_ref):
    vals = vals_ref[...]
    wts = wts_ref[...]
    pos = pos_ref[...]
    kf = jnp.float32(K)
    neg_inf = jnp.float32(-jnp.inf)

    rem = vals
    cum = jnp.float32(0.0)
    mrep = jnp.full((SLOTS, 128), jnp.float32(-3e38), jnp.float32)
    trep = jnp.zeros((SLOTS, 128), jnp.float32)
    row_iota = lax.broadcasted_iota(jnp.int32, (SLOTS, 128), 0)
    for k in range(K):
        m = jnp.max(rem)
        c = jnp.sum(jnp.where(rem == m, wts, 0.0))
        rem = jnp.where(rem == m, neg_inf, rem)
        take = jnp.minimum(c, jnp.maximum(kf - cum, 0.0))
        cum = cum + c
        rowmask = row_iota == k
        mrep = jnp.where(rowmask, jnp.maximum(m, jnp.float32(-3e38)), mrep)
        trep = jnp.where(rowmask, take, trep)
    big = SLOTS * 32
    repm = (lax.broadcasted_iota(jnp.int32, (big, SLOTS), 0) // 32
            == lax.broadcasted_iota(jnp.int32, (big, SLOTS), 1)).astype(jnp.float32)
    tile = (lax.broadcasted_iota(jnp.int32, (big, 32), 0) % 32
            == lax.broadcasted_iota(jnp.int32, (big, 32), 1)).astype(jnp.float32)
    vbig = jnp.dot(repm, mrep, preferred_element_type=jnp.float32)
    ubig = jnp.dot(repm, trep, preferred_element_type=jnp.float32)
    posbig = jnp.dot(tile, pos, preferred_element_type=jnp.float32)
    x = posbig - vbig
    bce = -jnp.log(jnp.clip(jax.nn.sigmoid(x), 1e-6, 1.0 - 1e-6))
    acc = jnp.sum(bce * ubig)
    out_ref[0, 0] = acc / jnp.float32(N_POS * N_NEG)


def kernel(score_neg, score_pos):
    cand_vals, cand_wts = _sc_topk()(score_neg)
    out = pl.pallas_call(
        _tc_reduce_body,
        out_shape=jax.ShapeDtypeStruct((1, 1), jnp.float32),
        out_specs=pl.BlockSpec(memory_space=pltpu.SMEM),
    )(
        cand_vals.reshape(NW * SLOTS // 128, 128),
        cand_wts.reshape(NW * SLOTS // 128, 128),
        score_pos.reshape(32, 128),
    )
    return out[0, 0]


# speedup vs baseline: 1.0266x; 1.0266x over previous
import functools

import jax
import jax.numpy as jnp
from jax import lax
from jax.experimental import pallas as pl
from jax.experimental.pallas import tpu as pltpu
from jax.experimental.pallas import tpu_sc as plsc

K = 50
N_NEG = 16384
N_POS = 4096
NC = 2
NS = 16
NW = NC * NS
CHUNK = N_NEG // NW
SLOTS = 64
LANES = 16


def _sc_topk_body(neg_hbm, vals_hbm, wts_hbm, chunk_v, key_v, vals_v, wts_v):
    wid = lax.axis_index("s") * NC + lax.axis_index("c")
    base = wid * CHUNK
    pltpu.sync_copy(neg_hbm.at[pl.ds(base, CHUNK)], chunk_v)

    neg_inf = jnp.float32(-jnp.inf)
    msb = jnp.int32(-2147483648)
    low31 = jnp.int32(2147483647)
    ones = jnp.ones((LANES,), jnp.float32)
    zeros = jnp.zeros((LANES,), jnp.float32)

    for i in range(CHUNK // LANES):
        b = plsc.bitcast(chunk_v[pl.ds(i * LANES, LANES)], jnp.int32)
        key_v[pl.ds(i * LANES, LANES)] = b ^ (
            lax.shift_right_arithmetic(b, 31) & low31)

    def rnd(r, p):
        bit = lax.shift_left(jnp.int32(1), 31 - r)
        t_u = p | bit
        t_s = t_u ^ msb
        cv = zeros
        for i in range(CHUNK // LANES):
            kk = key_v[pl.ds(i * LANES, LANES)]
            cv = cv + jnp.where(kk >= t_s, ones, zeros)
        return jnp.where(jnp.sum(cv) >= jnp.float32(K), t_u, p)

    t_s = lax.fori_loop(0, 32, rnd, jnp.int32(0)) ^ msb

    for g in range(SLOTS // LANES):
        vals_v[pl.ds(g * LANES, LANES)] = jnp.full((LANES,), neg_inf, jnp.float32)

    def emit(i, off):
        x = chunk_v[pl.ds(i * LANES, LANES)]
        sel = key_v[pl.ds(i * LANES, LANES)] > t_s
        ind = jnp.where(sel, jnp.int32(1), jnp.int32(0))
        incl = plsc.cumsum(ind)
        plsc.store_scatter(vals_v, [off + incl - ind], x, mask=sel)
        return off + jnp.sum(ind)

    off = lax.fori_loop(0, CHUNK // LANES, emit, jnp.int32(0))

    lane0 = lax.iota(jnp.int32, LANES) == 0
    t_vec = jnp.full((LANES,), t_s, jnp.int32)
    t_f = plsc.bitcast(
        t_vec ^ (lax.shift_right_arithmetic(t_vec, 31) & low31), jnp.float32)
    plsc.store_scatter(vals_v, [jnp.full((LANES,), off, jnp.int32)], t_f, mask=lane0)
    w_t = jnp.float32(K) - off.astype(jnp.float32)
    for g in range(SLOTS // LANES):
        idx16 = lax.iota(jnp.int32, LANES) + g * LANES
        wts_v[pl.ds(g * LANES, LANES)] = (
            jnp.where(idx16 < off, ones, zeros)
            + jnp.where(idx16 == off, jnp.full((LANES,), w_t, jnp.float32), zeros))
        v = vals_v[pl.ds(g * LANES, LANES)]
        vals_v[pl.ds(g * LANES, LANES)] = jnp.where(idx16 > off, neg_inf, v)

    pltpu.sync_copy(vals_v, vals_hbm.at[pl.ds(wid * SLOTS, SLOTS)])
    pltpu.sync_copy(wts_v, wts_hbm.at[pl.ds(wid * SLOTS, SLOTS)])


@functools.cache
def _sc_topk():
    return pl.kernel(
        _sc_topk_body,
        mesh=plsc.VectorSubcoreMesh(
            core_axis_name="c", subcore_axis_name="s",
            num_cores=NC, num_subcores=NS),
        out_type=(
            jax.ShapeDtypeStruct((NW * SLOTS,), jnp.float32),
            jax.ShapeDtypeStruct((NW * SLOTS,), jnp.float32),
        ),
        scratch_types=(
            pltpu.VMEM((CHUNK,), jnp.float32),
            pltpu.VMEM((CHUNK,), jnp.int32),
            pltpu.VMEM((SLOTS,), jnp.float32),
            pltpu.VMEM((SLOTS,), jnp.float32),
        ),
        compiler_params=pltpu.CompilerParams(needs_layout_passes=False),
    )


def _tc_reduce_body(vals_ref, wts_ref, pos_ref, out_ref):
    vals = vals_ref[...]
    wts = wts_ref[...]
    pos = pos_ref[...]
    kf = jnp.float32(K)
    neg_inf = jnp.float32(-jnp.inf)

    rem = vals
    mrep = jnp.full((SLOTS, 128), jnp.float32(-3e38), jnp.float32)
    row_iota = lax.broadcasted_iota(jnp.int32, (SLOTS, 128), 0)
    for k in range(K):
        m = jnp.max(rem)
        eqm = rem == m
        rem = jnp.where(eqm, neg_inf, rem)
        mrep = jnp.where(row_iota == k, jnp.maximum(m, jnp.float32(-3e38)), mrep)
    cnt = jnp.zeros((SLOTS, 128), jnp.float32)
    for r in range(vals.shape[0]):
        vrow = jnp.broadcast_to(vals[r:r + 1], (SLOTS, 128))
        wrow = jnp.broadcast_to(wts[r:r + 1], (SLOTS, 128))
        cnt = cnt + jnp.where(mrep == vrow, wrow, 0.0)
    ones_col = jnp.ones((128, 1), jnp.float32)
    c64 = jnp.dot(cnt, ones_col, preferred_element_type=jnp.float32)
    tri = (lax.broadcasted_iota(jnp.int32, (SLOTS, SLOTS), 0)
           > lax.broadcasted_iota(jnp.int32, (SLOTS, SLOTS), 1)).astype(jnp.float32)
    cum_excl = jnp.dot(tri, c64, preferred_element_type=jnp.float32)
    take64 = jnp.minimum(c64, jnp.maximum(kf - cum_excl, 0.0))
    trep = jnp.broadcast_to(take64, (SLOTS, 128))
    big = SLOTS * 32
    repm = (lax.broadcasted_iota(jnp.int32, (big, SLOTS), 0) // 32
            == lax.broadcasted_iota(jnp.int32, (big, SLOTS), 1)).astype(jnp.float32)
    tile = (lax.broadcasted_iota(jnp.int32, (big, 32), 0) % 32
            == lax.broadcasted_iota(jnp.int32, (big, 32), 1)).astype(jnp.float32)
    vbig = jnp.dot(repm, mrep, preferred_element_type=jnp.float32)
    ubig = jnp.dot(repm, trep, preferred_element_type=jnp.float32)
    posbig = jnp.dot(tile, pos, preferred_element_type=jnp.float32)
    x = posbig - vbig
    bce = -jnp.log(jnp.clip(jax.nn.sigmoid(x), 1e-6, 1.0 - 1e-6))
    acc = jnp.sum(bce * ubig)
    out_ref[0, 0] = acc / jnp.float32(N_POS * N_NEG)


def kernel(score_neg, score_pos):
    cand_vals, cand_wts = _sc_topk()(score_neg)
    out = pl.pallas_call(
        _tc_reduce_body,
        out_shape=jax.ShapeDtypeStruct((1, 1), jnp.float32),
        out_specs=pl.BlockSpec(memory_space=pltpu.SMEM),
    )(
        cand_vals.reshape(NW * SLOTS // 128, 128),
        cand_wts.reshape(NW * SLOTS // 128, 128),
        score_pos.reshape(32, 128),
    )
    return out[0, 0]
